# baseline (speedup 1.0000x reference)
import functools

import jax
import jax.numpy as jnp
from jax import lax
from jax.experimental import pallas as pl
from jax.experimental.pallas import tpu as pltpu
from jax.experimental.pallas import tpu_sc as plsc

B = 4096
LW = 200
LE = 50
D = 128
NC = 20

NWORK = 32
RPW = B // NWORK
LEP = 56
EG = 4
EN = EG * LEP
NG = RPW // EG


def _sc_gather_body(wids, eids, ww, we, sumw, vecent,
                    widxA, widxB, wrowsA, wrowsB,
                    eidxA, eidxB, erowsA, erowsB, srow,
                    semWA, semWB, semEA, semEB, semWrA, semWrB):
    c = lax.axis_index("c")
    s = lax.axis_index("s")
    wid = s * 2 + c
    base = wid * RPW

    def fire_w(row, idx_ref, rows_ref, sem):
        pltpu.sync_copy(wids.at[pl.ds(row * LW, LW)], idx_ref)
        pltpu.async_copy(ww.at[idx_ref.at[pl.ds(0, 128)]],
                         rows_ref.at[pl.ds(0, 128)], sem)
        pltpu.async_copy(ww.at[idx_ref.at[pl.ds(128, LW - 128)]],
                         rows_ref.at[pl.ds(128, LW - 128)], sem)

    def drain_w(rows_ref, sem):
        pltpu.make_async_copy(ww.at[pl.ds(0, LW)], rows_ref, sem).wait()

    def acc_store(rows_ref, row):
        def acc_body(j, acc):
            a = acc
            for u in range(4):
                a = tuple(a[k] + rows_ref[j * 4 + u, pl.ds(k * 16, 16)]
                          for k in range(8))
            return a
        acc = lax.fori_loop(0, LW // 4, acc_body,
                            tuple(jnp.zeros((16,), jnp.float32) for _ in range(8)))
        for k in range(8):
            srow[pl.ds(k * 16, 16)] = acc[k]
        pltpu.sync_copy(srow, sumw.at[row])

    fire_w(base, widxA, wrowsA, semWA)

    def word_body(i, carry):
        r0 = base + 2 * i
        fire_w(r0 + 1, widxB, wrowsB, semWB)
        drain_w(wrowsA, semWA)
        acc_store(wrowsA, r0)
        fire_w(jnp.minimum(r0 + 2, B - 1), widxA, wrowsA, semWA)
        drain_w(wrowsB, semWB)
        acc_store(wrowsB, r0 + 1)
        return carry

    lax.fori_loop(0, RPW // 2, word_body, 0)
    drain_w(wrowsA, semWA)

    def fire_e(g, idx_ref, rows_ref, sem):
        off = (base + g * EG) * LEP
        pltpu.sync_copy(eids.at[pl.ds(off, EN)], idx_ref)
        pltpu.async_copy(we.at[idx_ref.at[pl.ds(0, 128)]],
                         rows_ref.at[pl.ds(0, 128)], sem)
        pltpu.async_copy(we.at[idx_ref.at[pl.ds(128, EN - 128)]],
                         rows_ref.at[pl.ds(128, EN - 128)], sem)

    def drain_e(rows_ref, sem):
        pltpu.make_async_copy(we.at[pl.ds(0, EN)], rows_ref, sem).wait()

    def write_e(g, rows_ref, sem):
        off = (base + g * EG) * LEP
        pltpu.async_copy(rows_ref, vecent.at[pl.ds(off, EN)], sem)

    def drain_wr(rows_ref, sem):
        pltpu.make_async_copy(rows_ref, vecent.at[pl.ds(0, EN)], sem).wait()

    fire_e(0, eidxA, erowsA, semEA)
    fire_e(1, eidxB, erowsB, semEB)

    def ent_body(i, carry):
        g0 = 2 * i
        drain_e(erowsA, semEA)
        write_e(g0, erowsA, semWrA)
        drain_e(erowsB, semEB)
        write_e(g0 + 1, erowsB, semWrB)
        drain_wr(erowsA, semWrA)
        fire_e(jnp.minimum(g0 + 2, NG - 1), eidxA, erowsA, semEA)
        drain_wr(erowsB, semWrB)
        fire_e(jnp.minimum(g0 + 3, NG - 1), eidxB, erowsB, semEB)
        return carry

    lax.fori_loop(0, NG // 2, ent_body, 0)
    drain_e(erowsA, semEA)
    drain_e(erowsB, semEB)


@functools.cache
def _sc_gather_kernel():
    mesh = plsc.VectorSubcoreMesh(core_axis_name="c", subcore_axis_name="s")
    return pl.kernel(
        _sc_gather_body,
        mesh=mesh,
        out_type=[
            jax.ShapeDtypeStruct((B, D), jnp.float32),
            jax.ShapeDtypeStruct((B * LEP, D), jnp.float32),
        ],
        scratch_types=[
            pltpu.VMEM((LW,), jnp.int32),
            pltpu.VMEM((LW,), jnp.int32),
            pltpu.VMEM((LW, D), jnp.float32),
            pltpu.VMEM((LW, D), jnp.float32),
            pltpu.VMEM((EN,), jnp.int32),
            pltpu.VMEM((EN,), jnp.int32),
            pltpu.VMEM((EN, D), jnp.float32),
            pltpu.VMEM((EN, D), jnp.float32),
            pltpu.VMEM((D,), jnp.float32),
            pltpu.SemaphoreType.DMA,
            pltpu.SemaphoreType.DMA,
            pltpu.SemaphoreType.DMA,
            pltpu.SemaphoreType.DMA,
            pltpu.SemaphoreType.DMA,
            pltpu.SemaphoreType.DMA,
        ],
    )


BT = 256


def _tc_body(sw_ref, ve_ref, pp_ref, wid_ref, eid_ref, attw_ref, attb_ref,
             outw_ref, outb_ref, o_ref):
    sw = sw_ref[...]
    ve = ve_ref[...]
    lane = lax.broadcasted_iota(jnp.int32, (BT, LEP, 1), 1)
    ve = jnp.where(lane < LE, ve, 0.0)
    dn = jnp.maximum(jnp.sqrt(jnp.sum(sw * sw, axis=1, keepdims=True)), 1e-12)
    wn = sw / dn
    dn2 = jnp.maximum(jnp.sqrt(jnp.sum(ve * ve, axis=2)), 1e-12)
    cos = jnp.sum(wn[:, None, :] * ve, axis=2) / dn2
    w0 = attw_ref[0, 0]
    w1 = attw_ref[0, 1]
    bb = attb_ref[0, 0]
    logit = pp_ref[...] * w0 + cos * w1 + bb
    logit = jnp.where(eid_ref[...] == 0, -1e32, logit)
    m = jnp.max(logit, axis=1, keepdims=True)
    e = jnp.exp(logit - m)
    aw = e / jnp.sum(e, axis=1, keepdims=True)
    vf = jnp.sum(ve * aw[:, :, None], axis=1)
    cnt = jnp.sum((wid_ref[...] != 0).astype(jnp.float32), axis=1, keepdims=True)
    vf = vf + sw / cnt
    o_ref[...] = (jnp.dot(vf, outw_ref[...], preferred_element_type=jnp.float32)
                  + outb_ref[...])


def _tc_call(sumw, ve3, pp, wid, eid, attw, attb, outw, outb):
    return pl.pallas_call(
        _tc_body,
        grid=(B // BT,),
        in_specs=[
            pl.BlockSpec((BT, D), lambda i: (i, 0)),
            pl.BlockSpec((BT, LEP, D), lambda i: (i, 0, 0)),
            pl.BlockSpec((BT, LEP), lambda i: (i, 0)),
            pl.BlockSpec((BT, LW), lambda i: (i, 0)),
            pl.BlockSpec((BT, LEP), lambda i: (i, 0)),
            pl.BlockSpec((1, 2), lambda i: (0, 0)),
            pl.BlockSpec((1, 1), lambda i: (0, 0)),
            pl.BlockSpec((D, NC), lambda i: (0, 0)),
            pl.BlockSpec((1, NC), lambda i: (0, 0)),
        ],
        out_specs=pl.BlockSpec((BT, NC), lambda i: (i, 0)),
        out_shape=jax.ShapeDtypeStruct((B, NC), jnp.float32),
    )(sumw, ve3, pp, wid, eid, attw, attb, outw, outb)


def kernel(word_ids, entity_ids, prior_probs, W_word, W_entity, att_w, att_b,
           out_w, out_b):
    wids = word_ids.reshape(-1).astype(jnp.int32)
    eid_pad = jnp.pad(entity_ids.astype(jnp.int32), ((0, 0), (0, LEP - LE)))
    sumw, vecent = _sc_gather_kernel()(wids, eid_pad.reshape(-1), W_word, W_entity)
    ve3 = vecent.reshape(B, LEP, D)
    pp_p = jnp.pad(prior_probs, ((0, 0), (0, LEP - LE)))
    return _tc_call(
        sumw, ve3, pp_p,
        word_ids.astype(jnp.int32), eid_pad,
        att_w.reshape(1, 2).astype(jnp.float32),
        att_b.reshape(1, 1).astype(jnp.float32),
        out_w, out_b.reshape(1, NC),
    )


# speedup vs baseline: 3.4315x; 3.4315x over previous
import functools

import jax
import jax.numpy as jnp
from jax import lax
from jax.experimental import pallas as pl
from jax.experimental.pallas import tpu as pltpu
from jax.experimental.pallas import tpu_sc as plsc

B = 4096
LW = 200
LE = 50
D = 128
NC = 20

NWORK = 32
RPW = B // NWORK
LEP = 56
EG = 4
EN = EG * LEP
NG = RPW // EG


def _sc_gather_body(wids, eids, ww, we, sumw, vecent,
                    widxA, widxB, wrowsA, wrowsB,
                    eidxA, eidxB, erowsA, erowsB, srow,
                    semWA, semWB, semEA, semEB, semWrA, semWrB):
    c = lax.axis_index("c")
    s = lax.axis_index("s")
    wid = s * 2 + c
    base = wid * RPW

    def fire_w(row, idx_ref, rows_ref, sem):
        pltpu.sync_copy(wids.at[pl.ds(row * LW, LW)], idx_ref)
        pltpu.async_copy(ww.at[idx_ref.at[pl.ds(0, 128)]],
                         rows_ref.at[pl.ds(0, 128)], sem)
        pltpu.async_copy(ww.at[idx_ref.at[pl.ds(128, LW - 128)]],
                         rows_ref.at[pl.ds(128, LW - 128)], sem)

    def drain_w(rows_ref, sem):
        pltpu.make_async_copy(ww.at[pl.ds(0, LW)], rows_ref, sem).wait()

    def acc_store(rows_ref, row):
        def acc_body(j, acc):
            a = acc
            for u in range(4):
                a = tuple(a[k] + rows_ref[j * 4 + u, pl.ds(k * 16, 16)]
                          for k in range(8))
            return a
        acc = lax.fori_loop(0, LW // 4, acc_body,
                            tuple(jnp.zeros((16,), jnp.float32) for _ in range(8)))
        for k in range(8):
            srow[pl.ds(k * 16, 16)] = acc[k]
        pltpu.sync_copy(srow, sumw.at[row])

    fire_w(base, widxA, wrowsA, semWA)

    def word_body(i, carry):
        r0 = base + 2 * i
        fire_w(r0 + 1, widxB, wrowsB, semWB)
        drain_w(wrowsA, semWA)
        acc_store(wrowsA, r0)
        fire_w(jnp.minimum(r0 + 2, B - 1), widxA, wrowsA, semWA)
        drain_w(wrowsB, semWB)
        acc_store(wrowsB, r0 + 1)
        return carry

    lax.fori_loop(0, RPW // 2, word_body, 0)
    drain_w(wrowsA, semWA)

    def fire_e(g, idx_ref, rows_ref, sem):
        off = (base + g * EG) * LEP
        pltpu.sync_copy(eids.at[pl.ds(off, EN)], idx_ref)
        pltpu.async_copy(we.at[idx_ref.at[pl.ds(0, 128)]],
                         rows_ref.at[pl.ds(0, 128)], sem)
        pltpu.async_copy(we.at[idx_ref.at[pl.ds(128, EN - 128)]],
                         rows_ref.at[pl.ds(128, EN - 128)], sem)

    def drain_e(rows_ref, sem):
        pltpu.make_async_copy(we.at[pl.ds(0, EN)], rows_ref, sem).wait()

    def write_e(g, rows_ref, sem):
        off = (base + g * EG) * LEP
        pltpu.async_copy(rows_ref, vecent.at[pl.ds(off, EN)], sem)

    def drain_wr(rows_ref, sem):
        pltpu.make_async_copy(rows_ref, vecent.at[pl.ds(0, EN)], sem).wait()

    fire_e(0, eidxA, erowsA, semEA)
    fire_e(1, eidxB, erowsB, semEB)

    def ent_body(i, carry):
        g0 = 2 * i
        drain_e(erowsA, semEA)
        write_e(g0, erowsA, semWrA)
        drain_e(erowsB, semEB)
        write_e(g0 + 1, erowsB, semWrB)
        drain_wr(erowsA, semWrA)
        fire_e(jnp.minimum(g0 + 2, NG - 1), eidxA, erowsA, semEA)
        drain_wr(erowsB, semWrB)
        fire_e(jnp.minimum(g0 + 3, NG - 1), eidxB, erowsB, semEB)
        return carry

    lax.fori_loop(0, NG // 2, ent_body, 0)
    drain_e(erowsA, semEA)
    drain_e(erowsB, semEB)


@functools.cache
def _sc_gather_kernel():
    mesh = plsc.VectorSubcoreMesh(core_axis_name="c", subcore_axis_name="s")
    return pl.kernel(
        _sc_gather_body,
        mesh=mesh,
        out_type=[
            jax.ShapeDtypeStruct((B, D), jnp.float32),
            jax.ShapeDtypeStruct((B * LEP, D), jnp.float32),
        ],
        scratch_types=[
            pltpu.VMEM((LW,), jnp.int32),
            pltpu.VMEM((LW,), jnp.int32),
            pltpu.VMEM((LW, D), jnp.float32),
            pltpu.VMEM((LW, D), jnp.float32),
            pltpu.VMEM((EN,), jnp.int32),
            pltpu.VMEM((EN,), jnp.int32),
            pltpu.VMEM((EN, D), jnp.float32),
            pltpu.VMEM((EN, D), jnp.float32),
            pltpu.VMEM((D,), jnp.float32),
            pltpu.SemaphoreType.DMA,
            pltpu.SemaphoreType.DMA,
            pltpu.SemaphoreType.DMA,
            pltpu.SemaphoreType.DMA,
            pltpu.SemaphoreType.DMA,
            pltpu.SemaphoreType.DMA,
        ],
    )


BT = 256


def _tc_body(sw_ref, ve_ref, pp_ref, wid_ref, eid_ref, attw_ref, attb_ref,
             outw_ref, outb_ref, o_ref):
    sw = sw_ref[...]
    ve = ve_ref[...]
    lane = lax.broadcasted_iota(jnp.int32, (BT, LEP, 1), 1)
    ve = jnp.where(lane < LE, ve, 0.0)
    dn = jnp.maximum(jnp.sqrt(jnp.sum(sw * sw, axis=1, keepdims=True)), 1e-12)
    wn = sw / dn
    dn2 = jnp.maximum(jnp.sqrt(jnp.sum(ve * ve, axis=2)), 1e-12)
    cos = jnp.sum(wn[:, None, :] * ve, axis=2) / dn2
    w0 = attw_ref[0, 0]
    w1 = attw_ref[0, 1]
    bb = attb_ref[0, 0]
    logit = pp_ref[...] * w0 + cos * w1 + bb
    lane2 = lax.broadcasted_iota(jnp.int32, (BT, LEP), 1)
    logit = jnp.where((eid_ref[...] == 0) | (lane2 >= LE), -1e32, logit)
    m = jnp.max(logit, axis=1, keepdims=True)
    e = jnp.exp(logit - m)
    aw = e / jnp.sum(e, axis=1, keepdims=True)
    vf = jnp.sum(ve * aw[:, :, None], axis=1)
    cnt = jnp.sum((wid_ref[...] != 0).astype(jnp.float32), axis=1, keepdims=True)
    vf = vf + sw / cnt
    o_ref[...] = (jnp.dot(vf, outw_ref[...], preferred_element_type=jnp.float32)
                  + outb_ref[...])


def _tc_call(sumw, ve3, pp, wid, eid, attw, attb, outw, outb):
    return pl.pallas_call(
        _tc_body,
        grid=(B // BT,),
        in_specs=[
            pl.BlockSpec((BT, D), lambda i: (i, 0)),
            pl.BlockSpec((BT, LEP, D), lambda i: (i, 0, 0)),
            pl.BlockSpec((BT, LEP), lambda i: (i, 0)),
            pl.BlockSpec((BT, LW), lambda i: (i, 0)),
            pl.BlockSpec((BT, LEP), lambda i: (i, 0)),
            pl.BlockSpec((1, 2), lambda i: (0, 0)),
            pl.BlockSpec((1, 1), lambda i: (0, 0)),
            pl.BlockSpec((D, NC), lambda i: (0, 0)),
            pl.BlockSpec((1, NC), lambda i: (0, 0)),
        ],
        out_specs=pl.BlockSpec((BT, NC), lambda i: (i, 0)),
        out_shape=jax.ShapeDtypeStruct((B, NC), jnp.float32),
    )(sumw, ve3, pp, wid, eid, attw, attb, outw, outb)


def kernel(word_ids, entity_ids, prior_probs, W_word, W_entity, att_w, att_b,
           out_w, out_b):
    wids = word_ids.reshape(-1).astype(jnp.int32)
    spread = (lax.broadcasted_iota(jnp.int32, (B, LEP - LE), 0) * (LEP - LE)
              + lax.broadcasted_iota(jnp.int32, (B, LEP - LE), 1))
    eid_pad = jnp.concatenate([entity_ids.astype(jnp.int32), spread], axis=1)
    sumw, vecent = _sc_gather_kernel()(wids, eid_pad.reshape(-1), W_word, W_entity)
    ve3 = vecent.reshape(B, LEP, D)
    pp_p = jnp.pad(prior_probs, ((0, 0), (0, LEP - LE)))
    return _tc_call(
        sumw, ve3, pp_p,
        word_ids.astype(jnp.int32), eid_pad,
        att_w.reshape(1, 2).astype(jnp.float32),
        att_b.reshape(1, 1).astype(jnp.float32),
        out_w, out_b.reshape(1, NC),
    )


# speedup vs baseline: 3.8277x; 1.1155x over previous
import functools

import jax
import jax.numpy as jnp
from jax import lax
from jax.experimental import pallas as pl
from jax.experimental.pallas import tpu as pltpu
from jax.experimental.pallas import tpu_sc as plsc

B = 4096
LW = 200
LE = 50
D = 128
NC = 20

NWORK = 32
LEP = 56
EG = 4
EN = EG * LEP
NCH = 2
CB = B // NCH
RPW = CB // NWORK
NG = RPW // EG
BT = 256


def _sc_gather_body(wids, eids, ww, we, sumw, vecent,
                    widxA, widxB, wrowsA, wrowsB,
                    eidxA, eidxB, erowsA, erowsB, srow,
                    semWA, semWB, semEA, semEB, semWrA, semWrB):
    c = lax.axis_index("c")
    s = lax.axis_index("s")
    wid = s * 2 + c
    base = wid * RPW

    def fire_w(row, idx_ref, rows_ref, sem):
        pltpu.sync_copy(wids.at[pl.ds(row * LW, LW)], idx_ref)
        pltpu.async_copy(ww.at[idx_ref.at[pl.ds(0, 128)]],
                         rows_ref.at[pl.ds(0, 128)], sem)
        pltpu.async_copy(ww.at[idx_ref.at[pl.ds(128, LW - 128)]],
                         rows_ref.at[pl.ds(128, LW - 128)], sem)

    def drain_w(rows_ref, sem):
        pltpu.make_async_copy(ww.at[pl.ds(0, LW)], rows_ref, sem).wait()

    def acc_store(rows_ref, row):
        def acc_body(j, acc):
            a = acc
            for u in range(4):
                a = tuple(a[k] + rows_ref[j * 4 + u, pl.ds(k * 16, 16)]
                          for k in range(8))
            return a
        acc = lax.fori_loop(0, LW // 4, acc_body,
                            tuple(jnp.zeros((16,), jnp.float32) for _ in range(8)))
        for k in range(8):
            srow[pl.ds(k * 16, 16)] = acc[k]
        pltpu.sync_copy(srow, sumw.at[row])

    fire_w(base, widxA, wrowsA, semWA)

    def word_body(i, carry):
        r0 = base + 2 * i
        fire_w(r0 + 1, widxB, wrowsB, semWB)
        drain_w(wrowsA, semWA)
        acc_store(wrowsA, r0)
        fire_w(jnp.minimum(r0 + 2, CB - 1), widxA, wrowsA, semWA)
        drain_w(wrowsB, semWB)
        acc_store(wrowsB, r0 + 1)
        return carry

    lax.fori_loop(0, RPW // 2, word_body, 0)
    drain_w(wrowsA, semWA)

    def fire_e(g, idx_ref, rows_ref, sem):
        off = (base + g * EG) * LEP
        pltpu.sync_copy(eids.at[pl.ds(off, EN)], idx_ref)
        pltpu.async_copy(we.at[idx_ref.at[pl.ds(0, 128)]],
                         rows_ref.at[pl.ds(0, 128)], sem)
        pltpu.async_copy(we.at[idx_ref.at[pl.ds(128, EN - 128)]],
                         rows_ref.at[pl.ds(128, EN - 128)], sem)

    def drain_e(rows_ref, sem):
        pltpu.make_async_copy(we.at[pl.ds(0, EN)], rows_ref, sem).wait()

    def write_e(g, rows_ref, sem):
        off = (base + g * EG) * LEP
        pltpu.async_copy(rows_ref, vecent.at[pl.ds(off, EN)], sem)

    def drain_wr(rows_ref, sem):
        pltpu.make_async_copy(rows_ref, vecent.at[pl.ds(0, EN)], sem).wait()

    fire_e(0, eidxA, erowsA, semEA)
    fire_e(1, eidxB, erowsB, semEB)

    def ent_body(i, carry):
        g0 = 2 * i
        drain_e(erowsA, semEA)
        write_e(g0, erowsA, semWrA)
        drain_e(erowsB, semEB)
        write_e(g0 + 1, erowsB, semWrB)
        drain_wr(erowsA, semWrA)
        fire_e(jnp.minimum(g0 + 2, NG - 1), eidxA, erowsA, semEA)
        drain_wr(erowsB, semWrB)
        fire_e(jnp.minimum(g0 + 3, NG - 1), eidxB, erowsB, semEB)
        return carry

    lax.fori_loop(0, NG // 2, ent_body, 0)
    drain_e(erowsA, semEA)
    drain_e(erowsB, semEB)


@functools.cache
def _sc_gather_kernel():
    mesh = plsc.VectorSubcoreMesh(core_axis_name="c", subcore_axis_name="s")
    return pl.kernel(
        _sc_gather_body,
        mesh=mesh,
        out_type=[
            jax.ShapeDtypeStruct((CB, D), jnp.float32),
            jax.ShapeDtypeStruct((CB * LEP, D), jnp.float32),
        ],
        scratch_types=[
            pltpu.VMEM((LW,), jnp.int32),
            pltpu.VMEM((LW,), jnp.int32),
            pltpu.VMEM((LW, D), jnp.float32),
            pltpu.VMEM((LW, D), jnp.float32),
            pltpu.VMEM((EN,), jnp.int32),
            pltpu.VMEM((EN,), jnp.int32),
            pltpu.VMEM((EN, D), jnp.float32),
            pltpu.VMEM((EN, D), jnp.float32),
            pltpu.VMEM((D,), jnp.float32),
            pltpu.SemaphoreType.DMA,
            pltpu.SemaphoreType.DMA,
            pltpu.SemaphoreType.DMA,
            pltpu.SemaphoreType.DMA,
            pltpu.SemaphoreType.DMA,
            pltpu.SemaphoreType.DMA,
        ],
    )


def _tc_body(sw_ref, ve_ref, pp_ref, wid_ref, eid_ref, attw_ref, attb_ref,
             outw_ref, outb_ref, o_ref):
    sw = sw_ref[...]
    ve = ve_ref[...]
    lane = lax.broadcasted_iota(jnp.int32, (BT, LEP, 1), 1)
    ve = jnp.where(lane < LE, ve, 0.0)
    dn = jnp.maximum(jnp.sqrt(jnp.sum(sw * sw, axis=1, keepdims=True)), 1e-12)
    wn = sw / dn
    dn2 = jnp.maximum(jnp.sqrt(jnp.sum(ve * ve, axis=2)), 1e-12)
    cos = jnp.sum(wn[:, None, :] * ve, axis=2) / dn2
    w0 = attw_ref[0, 0]
    w1 = attw_ref[0, 1]
    bb = attb_ref[0, 0]
    logit = pp_ref[...] * w0 + cos * w1 + bb
    lane2 = lax.broadcasted_iota(jnp.int32, (BT, LEP), 1)
    logit = jnp.where((eid_ref[...] == 0) | (lane2 >= LE), -1e32, logit)
    m = jnp.max(logit, axis=1, keepdims=True)
    e = jnp.exp(logit - m)
    aw = e / jnp.sum(e, axis=1, keepdims=True)
    vf = jnp.sum(ve * aw[:, :, None], axis=1)
    cnt = jnp.sum((wid_ref[...] != 0).astype(jnp.float32), axis=1, keepdims=True)
    vf = vf + sw / cnt
    o_ref[...] = (jnp.dot(vf, outw_ref[...], preferred_element_type=jnp.float32)
                  + outb_ref[...])


def _tc_call(sumw, ve3, pp, wid, eid, attw, attb, outw, outb):
    return pl.pallas_call(
        _tc_body,
        grid=(CB // BT,),
        in_specs=[
            pl.BlockSpec((BT, D), lambda i: (i, 0)),
            pl.BlockSpec((BT, LEP, D), lambda i: (i, 0, 0)),
            pl.BlockSpec((BT, LEP), lambda i: (i, 0)),
            pl.BlockSpec((BT, LW), lambda i: (i, 0)),
            pl.BlockSpec((BT, LEP), lambda i: (i, 0)),
            pl.BlockSpec((1, 2), lambda i: (0, 0)),
            pl.BlockSpec((1, 1), lambda i: (0, 0)),
            pl.BlockSpec((D, NC), lambda i: (0, 0)),
            pl.BlockSpec((1, NC), lambda i: (0, 0)),
        ],
        out_specs=pl.BlockSpec((BT, NC), lambda i: (i, 0)),
        out_shape=jax.ShapeDtypeStruct((CB, NC), jnp.float32),
    )(sumw, ve3, pp, wid, eid, attw, attb, outw, outb)


def kernel(word_ids, entity_ids, prior_probs, W_word, W_entity, att_w, att_b,
           out_w, out_b):
    word_ids = word_ids.astype(jnp.int32)
    spread = (lax.broadcasted_iota(jnp.int32, (B, LEP - LE), 0) * (LEP - LE)
              + lax.broadcasted_iota(jnp.int32, (B, LEP - LE), 1))
    eid_pad = jnp.concatenate([entity_ids.astype(jnp.int32), spread], axis=1)
    pp_p = jnp.pad(prior_probs, ((0, 0), (0, LEP - LE)))
    attw = att_w.reshape(1, 2).astype(jnp.float32)
    attb = att_b.reshape(1, 1).astype(jnp.float32)
    outb = out_b.reshape(1, NC)

    outs = []
    for ch in range(NCH):
        rows = slice(ch * CB, (ch + 1) * CB)
        sumw, vecent = _sc_gather_kernel()(
            word_ids[rows].reshape(-1),
            eid_pad[rows].reshape(-1),
            W_word, W_entity)
        outs.append(_tc_call(
            sumw, vecent.reshape(CB, LEP, D), pp_p[rows],
            word_ids[rows], eid_pad[rows],
            attw, attb, out_w, outb,
        ))
    return jnp.concatenate(outs, axis=0)


# speedup vs baseline: 3.8773x; 1.0129x over previous
import functools

import jax
import jax.numpy as jnp
from jax import lax
from jax.experimental import pallas as pl
from jax.experimental.pallas import tpu as pltpu
from jax.experimental.pallas import tpu_sc as plsc

B = 4096
LW = 200
LE = 50
D = 128
NC = 20

NWORK = 32
LEP = 56
EG = 4
EN = EG * LEP
NCH = 4
CB = B // NCH
RPW = CB // NWORK
NG = RPW // EG
BT = 256


def _sc_gather_body(wids, eids, ww, we, sumw, vecent,
                    widxA, widxB, wrowsA, wrowsB,
                    eidxA, eidxB, erowsA, erowsB, srow,
                    semWA, semWB, semEA, semEB, semWrA, semWrB):
    c = lax.axis_index("c")
    s = lax.axis_index("s")
    wid = s * 2 + c
    base = wid * RPW

    def fire_w(row, idx_ref, rows_ref, sem):
        pltpu.sync_copy(wids.at[pl.ds(row * LW, LW)], idx_ref)
        pltpu.async_copy(ww.at[idx_ref.at[pl.ds(0, 128)]],
                         rows_ref.at[pl.ds(0, 128)], sem)
        pltpu.async_copy(ww.at[idx_ref.at[pl.ds(128, LW - 128)]],
                         rows_ref.at[pl.ds(128, LW - 128)], sem)

    def drain_w(rows_ref, sem):
        pltpu.make_async_copy(ww.at[pl.ds(0, LW)], rows_ref, sem).wait()

    def acc_store(rows_ref, row):
        def acc_body(j, acc):
            a = acc
            for u in range(4):
                a = tuple(a[k] + rows_ref[j * 4 + u, pl.ds(k * 16, 16)]
                          for k in range(8))
            return a
        acc = lax.fori_loop(0, LW // 4, acc_body,
                            tuple(jnp.zeros((16,), jnp.float32) for _ in range(8)))
        for k in range(8):
            srow[pl.ds(k * 16, 16)] = acc[k]
        pltpu.sync_copy(srow, sumw.at[row])

    fire_w(base, widxA, wrowsA, semWA)

    def word_body(i, carry):
        r0 = base + 2 * i
        fire_w(r0 + 1, widxB, wrowsB, semWB)
        drain_w(wrowsA, semWA)
        acc_store(wrowsA, r0)
        fire_w(jnp.minimum(r0 + 2, CB - 1), widxA, wrowsA, semWA)
        drain_w(wrowsB, semWB)
        acc_store(wrowsB, r0 + 1)
        return carry

    lax.fori_loop(0, RPW // 2, word_body, 0)
    drain_w(wrowsA, semWA)

    def fire_e(g, idx_ref, rows_ref, sem):
        off = (base + g * EG) * LEP
        pltpu.sync_copy(eids.at[pl.ds(off, EN)], idx_ref)
        pltpu.async_copy(we.at[idx_ref.at[pl.ds(0, 128)]],
                         rows_ref.at[pl.ds(0, 128)], sem)
        pltpu.async_copy(we.at[idx_ref.at[pl.ds(128, EN - 128)]],
                         rows_ref.at[pl.ds(128, EN - 128)], sem)

    def drain_e(rows_ref, sem):
        pltpu.make_async_copy(we.at[pl.ds(0, EN)], rows_ref, sem).wait()

    def write_e(g, rows_ref, sem):
        off = (base + g * EG) * LEP
        pltpu.async_copy(rows_ref, vecent.at[pl.ds(off, EN)], sem)

    def drain_wr(rows_ref, sem):
        pltpu.make_async_copy(rows_ref, vecent.at[pl.ds(0, EN)], sem).wait()

    fire_e(0, eidxA, erowsA, semEA)
    fire_e(1, eidxB, erowsB, semEB)

    def ent_body(i, carry):
        g0 = 2 * i
        drain_e(erowsA, semEA)
        write_e(g0, erowsA, semWrA)
        drain_e(erowsB, semEB)
        write_e(g0 + 1, erowsB, semWrB)
        drain_wr(erowsA, semWrA)
        fire_e(jnp.minimum(g0 + 2, NG - 1), eidxA, erowsA, semEA)
        drain_wr(erowsB, semWrB)
        fire_e(jnp.minimum(g0 + 3, NG - 1), eidxB, erowsB, semEB)
        return carry

    lax.fori_loop(0, NG // 2, ent_body, 0)
    drain_e(erowsA, semEA)
    drain_e(erowsB, semEB)


@functools.cache
def _sc_gather_kernel():
    mesh = plsc.VectorSubcoreMesh(core_axis_name="c", subcore_axis_name="s")
    return pl.kernel(
        _sc_gather_body,
        mesh=mesh,
        out_type=[
            jax.ShapeDtypeStruct((CB, D), jnp.float32),
            jax.ShapeDtypeStruct((CB * LEP, D), jnp.float32),
        ],
        scratch_types=[
            pltpu.VMEM((LW,), jnp.int32),
            pltpu.VMEM((LW,), jnp.int32),
            pltpu.VMEM((LW, D), jnp.float32),
            pltpu.VMEM((LW, D), jnp.float32),
            pltpu.VMEM((EN,), jnp.int32),
            pltpu.VMEM((EN,), jnp.int32),
            pltpu.VMEM((EN, D), jnp.float32),
            pltpu.VMEM((EN, D), jnp.float32),
            pltpu.VMEM((D,), jnp.float32),
            pltpu.SemaphoreType.DMA,
            pltpu.SemaphoreType.DMA,
            pltpu.SemaphoreType.DMA,
            pltpu.SemaphoreType.DMA,
            pltpu.SemaphoreType.DMA,
            pltpu.SemaphoreType.DMA,
        ],
    )


def _tc_body(sw_ref, ve_ref, pp_ref, wid_ref, eid_ref, attw_ref, attb_ref,
             outw_ref, outb_ref, o_ref):
    sw = sw_ref[...]
    ve = ve_ref[...]
    lane = lax.broadcasted_iota(jnp.int32, (BT, LEP, 1), 1)
    ve = jnp.where(lane < LE, ve, 0.0)
    dn = jnp.maximum(jnp.sqrt(jnp.sum(sw * sw, axis=1, keepdims=True)), 1e-12)
    wn = sw / dn
    dn2 = jnp.maximum(jnp.sqrt(jnp.sum(ve * ve, axis=2)), 1e-12)
    cos = jnp.sum(wn[:, None, :] * ve, axis=2) / dn2
    w0 = attw_ref[0, 0]
    w1 = attw_ref[0, 1]
    bb = attb_ref[0, 0]
    logit = pp_ref[...] * w0 + cos * w1 + bb
    lane2 = lax.broadcasted_iota(jnp.int32, (BT, LEP), 1)
    logit = jnp.where((eid_ref[...] == 0) | (lane2 >= LE), -1e32, logit)
    m = jnp.max(logit, axis=1, keepdims=True)
    e = jnp.exp(logit - m)
    aw = e / jnp.sum(e, axis=1, keepdims=True)
    vf = jnp.sum(ve * aw[:, :, None], axis=1)
    cnt = jnp.sum((wid_ref[...] != 0).astype(jnp.float32), axis=1, keepdims=True)
    vf = vf + sw / cnt
    o_ref[...] = (jnp.dot(vf, outw_ref[...], preferred_element_type=jnp.float32)
                  + outb_ref[...])


def _tc_call(sumw, ve3, pp, wid, eid, attw, attb, outw, outb):
    return pl.pallas_call(
        _tc_body,
        grid=(CB // BT,),
        in_specs=[
            pl.BlockSpec((BT, D), lambda i: (i, 0)),
            pl.BlockSpec((BT, LEP, D), lambda i: (i, 0, 0)),
            pl.BlockSpec((BT, LEP), lambda i: (i, 0)),
            pl.BlockSpec((BT, LW), lambda i: (i, 0)),
            pl.BlockSpec((BT, LEP), lambda i: (i, 0)),
            pl.BlockSpec((1, 2), lambda i: (0, 0)),
            pl.BlockSpec((1, 1), lambda i: (0, 0)),
            pl.BlockSpec((D, NC), lambda i: (0, 0)),
            pl.BlockSpec((1, NC), lambda i: (0, 0)),
        ],
        out_specs=pl.BlockSpec((BT, NC), lambda i: (i, 0)),
        out_shape=jax.ShapeDtypeStruct((CB, NC), jnp.float32),
    )(sumw, ve3, pp, wid, eid, attw, attb, outw, outb)


def kernel(word_ids, entity_ids, prior_probs, W_word, W_entity, att_w, att_b,
           out_w, out_b):
    word_ids = word_ids.astype(jnp.int32)
    spread = (lax.broadcasted_iota(jnp.int32, (B, LEP - LE), 0) * (LEP - LE)
              + lax.broadcasted_iota(jnp.int32, (B, LEP - LE), 1))
    eid_pad = jnp.concatenate([entity_ids.astype(jnp.int32), spread], axis=1)
    pp_p = jnp.pad(prior_probs, ((0, 0), (0, LEP - LE)))
    attw = att_w.reshape(1, 2).astype(jnp.float32)
    attb = att_b.reshape(1, 1).astype(jnp.float32)
    outb = out_b.reshape(1, NC)

    outs = []
    for ch in range(NCH):
        rows = slice(ch * CB, (ch + 1) * CB)
        sumw, vecent = _sc_gather_kernel()(
            word_ids[rows].reshape(-1),
            eid_pad[rows].reshape(-1),
            W_word, W_entity)
        outs.append(_tc_call(
            sumw, vecent.reshape(CB, LEP, D), pp_p[rows],
            word_ids[rows], eid_pad[rows],
            attw, attb, out_w, outb,
        ))
    return jnp.concatenate(outs, axis=0)


# speedup vs baseline: 4.1473x; 1.0696x over previous
import functools

import jax
import jax.numpy as jnp
from jax import lax
from jax.experimental import pallas as pl
from jax.experimental.pallas import tpu as pltpu
from jax.experimental.pallas import tpu_sc as plsc

B = 4096
LW = 200
LE = 50
D = 128
NC = 20

NWORK = 32
LEP = 56
EG = 4
EN = EG * LEP
NCH = 4
CB = B // NCH
RPW = CB // NWORK
NG = RPW // EG
BT = 256


def _sc_gather_body(wids, eids, ww, we, sumw, vecent,
                    widxA, widxB, wrowsA, wrowsB,
                    eidxA, eidxB, erowsA, erowsB, srow,
                    semWA, semWB, semEA, semEB, semWrA, semWrB):
    c = lax.axis_index("c")
    s = lax.axis_index("s")
    wid = s * 2 + c
    base = wid * RPW

    def fire_w(row, idx_ref, rows_ref, sem):
        pltpu.sync_copy(wids.at[pl.ds(row * LW, LW)], idx_ref)
        pltpu.async_copy(ww.at[idx_ref.at[pl.ds(0, 128)]],
                         rows_ref.at[pl.ds(0, 128)], sem)
        pltpu.async_copy(ww.at[idx_ref.at[pl.ds(128, LW - 128)]],
                         rows_ref.at[pl.ds(128, LW - 128)], sem)

    def drain_w(rows_ref, sem):
        pltpu.make_async_copy(ww.at[pl.ds(0, LW)], rows_ref, sem).wait()

    def acc_store(rows_ref, row):
        def acc_body(j, acc):
            a = acc
            for u in range(4):
                a = tuple(a[k] + rows_ref[j * 4 + u, pl.ds(k * 16, 16)]
                          for k in range(8))
            return a
        acc = lax.fori_loop(0, LW // 4, acc_body,
                            tuple(jnp.zeros((16,), jnp.float32) for _ in range(8)))
        for k in range(8):
            srow[pl.ds(k * 16, 16)] = acc[k]
        pltpu.sync_copy(srow, sumw.at[row])

    def fire_e(g, idx_ref, rows_ref, sem):
        off = (base + g * EG) * LEP
        pltpu.sync_copy(eids.at[pl.ds(off, EN)], idx_ref)
        pltpu.async_copy(we.at[idx_ref.at[pl.ds(0, 128)]],
                         rows_ref.at[pl.ds(0, 128)], sem)
        pltpu.async_copy(we.at[idx_ref.at[pl.ds(128, EN - 128)]],
                         rows_ref.at[pl.ds(128, EN - 128)], sem)

    def drain_e(rows_ref, sem):
        pltpu.make_async_copy(we.at[pl.ds(0, EN)], rows_ref, sem).wait()

    def write_e(g, rows_ref, sem):
        off = (base + g * EG) * LEP
        pltpu.async_copy(rows_ref, vecent.at[pl.ds(off, EN)], sem)

    def drain_wr(rows_ref, sem):
        pltpu.make_async_copy(rows_ref, vecent.at[pl.ds(0, EN)], sem).wait()

    fire_w(base, widxA, wrowsA, semWA)
    fire_e(0, eidxA, erowsA, semEA)
    fire_e(1, eidxB, erowsB, semEB)

    def super_body(i, carry):
        g0 = 2 * i
        for j in range(4):
            rj = base + 8 * i + 2 * j
            fire_w(rj + 1, widxB, wrowsB, semWB)
            drain_w(wrowsA, semWA)
            acc_store(wrowsA, rj)
            fire_w(jnp.minimum(rj + 2, CB - 1), widxA, wrowsA, semWA)
            drain_w(wrowsB, semWB)
            acc_store(wrowsB, rj + 1)
            if j == 0:
                drain_e(erowsA, semEA)
                write_e(g0, erowsA, semWrA)
            elif j == 1:
                drain_e(erowsB, semEB)
                write_e(g0 + 1, erowsB, semWrB)
            elif j == 2:
                drain_wr(erowsA, semWrA)
                fire_e(jnp.minimum(g0 + 2, NG - 1), eidxA, erowsA, semEA)
            else:
                drain_wr(erowsB, semWrB)
                fire_e(jnp.minimum(g0 + 3, NG - 1), eidxB, erowsB, semEB)
        return carry

    lax.fori_loop(0, RPW // 8, super_body, 0)
    drain_w(wrowsA, semWA)
    drain_e(erowsA, semEA)
    drain_e(erowsB, semEB)


@functools.cache
def _sc_gather_kernel():
    mesh = plsc.VectorSubcoreMesh(core_axis_name="c", subcore_axis_name="s")
    return pl.kernel(
        _sc_gather_body,
        mesh=mesh,
        out_type=[
            jax.ShapeDtypeStruct((CB, D), jnp.float32),
            jax.ShapeDtypeStruct((CB * LEP, D), jnp.float32),
        ],
        scratch_types=[
            pltpu.VMEM((LW,), jnp.int32),
            pltpu.VMEM((LW,), jnp.int32),
            pltpu.VMEM((LW, D), jnp.float32),
            pltpu.VMEM((LW, D), jnp.float32),
            pltpu.VMEM((EN,), jnp.int32),
            pltpu.VMEM((EN,), jnp.int32),
            pltpu.VMEM((EN, D), jnp.float32),
            pltpu.VMEM((EN, D), jnp.float32),
            pltpu.VMEM((D,), jnp.float32),
            pltpu.SemaphoreType.DMA,
            pltpu.SemaphoreType.DMA,
            pltpu.SemaphoreType.DMA,
            pltpu.SemaphoreType.DMA,
            pltpu.SemaphoreType.DMA,
            pltpu.SemaphoreType.DMA,
        ],
    )


def _tc_body(sw_ref, ve_ref, pp_ref, wid_ref, eid_ref, attw_ref, attb_ref,
             outw_ref, outb_ref, o_ref):
    sw = sw_ref[...]
    ve = ve_ref[...]
    lane = lax.broadcasted_iota(jnp.int32, (BT, LEP, 1), 1)
    ve = jnp.where(lane < LE, ve, 0.0)
    dn = jnp.maximum(jnp.sqrt(jnp.sum(sw * sw, axis=1, keepdims=True)), 1e-12)
    wn = sw / dn
    dn2 = jnp.maximum(jnp.sqrt(jnp.sum(ve * ve, axis=2)), 1e-12)
    cos = jnp.sum(wn[:, None, :] * ve, axis=2) / dn2
    w0 = attw_ref[0, 0]
    w1 = attw_ref[0, 1]
    bb = attb_ref[0, 0]
    logit = pp_ref[...] * w0 + cos * w1 + bb
    lane2 = lax.broadcasted_iota(jnp.int32, (BT, LEP), 1)
    logit = jnp.where((eid_ref[...] == 0) | (lane2 >= LE), -1e32, logit)
    m = jnp.max(logit, axis=1, keepdims=True)
    e = jnp.exp(logit - m)
    aw = e / jnp.sum(e, axis=1, keepdims=True)
    vf = jnp.sum(ve * aw[:, :, None], axis=1)
    cnt = jnp.sum((wid_ref[...] != 0).astype(jnp.float32), axis=1, keepdims=True)
    vf = vf + sw / cnt
    o_ref[...] = (jnp.dot(vf, outw_ref[...], preferred_element_type=jnp.float32)
                  + outb_ref[...])


def _tc_call(sumw, ve3, pp, wid, eid, attw, attb, outw, outb):
    return pl.pallas_call(
        _tc_body,
        grid=(CB // BT,),
        in_specs=[
            pl.BlockSpec((BT, D), lambda i: (i, 0)),
            pl.BlockSpec((BT, LEP, D), lambda i: (i, 0, 0)),
            pl.BlockSpec((BT, LEP), lambda i: (i, 0)),
            pl.BlockSpec((BT, LW), lambda i: (i, 0)),
            pl.BlockSpec((BT, LEP), lambda i: (i, 0)),
            pl.BlockSpec((1, 2), lambda i: (0, 0)),
            pl.BlockSpec((1, 1), lambda i: (0, 0)),
            pl.BlockSpec((D, NC), lambda i: (0, 0)),
            pl.BlockSpec((1, NC), lambda i: (0, 0)),
        ],
        out_specs=pl.BlockSpec((BT, NC), lambda i: (i, 0)),
        out_shape=jax.ShapeDtypeStruct((CB, NC), jnp.float32),
    )(sumw, ve3, pp, wid, eid, attw, attb, outw, outb)


def kernel(word_ids, entity_ids, prior_probs, W_word, W_entity, att_w, att_b,
           out_w, out_b):
    word_ids = word_ids.astype(jnp.int32)
    spread = (lax.broadcasted_iota(jnp.int32, (B, LEP - LE), 0) * (LEP - LE)
              + lax.broadcasted_iota(jnp.int32, (B, LEP - LE), 1))
    eid_pad = jnp.concatenate([entity_ids.astype(jnp.int32), spread], axis=1)
    pp_p = jnp.pad(prior_probs, ((0, 0), (0, LEP - LE)))
    attw = att_w.reshape(1, 2).astype(jnp.float32)
    attb = att_b.reshape(1, 1).astype(jnp.float32)
    outb = out_b.reshape(1, NC)

    outs = []
    for ch in range(NCH):
        rows = slice(ch * CB, (ch + 1) * CB)
        sumw, vecent = _sc_gather_kernel()(
            word_ids[rows].reshape(-1),
            eid_pad[rows].reshape(-1),
            W_word, W_entity)
        outs.append(_tc_call(
            sumw, vecent.reshape(CB, LEP, D), pp_p[rows],
            word_ids[rows], eid_pad[rows],
            attw, attb, out_w, outb,
        ))
    return jnp.concatenate(outs, axis=0)
